# Initial kernel scaffold; baseline (speedup 1.0000x reference)
import jax
import jax.numpy as jnp
from jax.experimental import pallas as pl


def kernel(x, edge_index, edge_attr, batch, W_enc, b_enc, eps_all, W_edge_all, b_edge_all, W1_all, b1_all, W2_all, b2_all, W_fc, b_fc):
    raise NotImplementedError("write your pallas kernel here")


# speedup vs baseline: 3.5774x; 3.5774x over previous
import functools

import jax
import jax.numpy as jnp
from jax import lax
from jax.experimental import pallas as pl
from jax.experimental.pallas import tpu as pltpu
from jax.experimental.pallas import tpu_sc as plsc

_N = 10000
_E = 320000
_D = 128
_EDGE_D = 16
_G = 64
_L = 3
_OUT = 128

_NC = 2
_NS = 16
_NT = _NC * _NS
_B = 80
_CH = _E // (_NT * _B)
_IG = 25
_RPT = 624
_XB = 16
_LANES = 16


def _sc_agg_body(h_hbm, e_hbm, src_hbm, dst_hbm, out_hbm,
                 src_v, dst_v, rows_v, msg_v, zbuf, agg_sh, sem_g, sem_e):
    c = lax.axis_index("c")
    s = lax.axis_index("s")
    tid = c * _NS + s

    zv = jnp.zeros((_LANES,), jnp.float32)

    def _zrow(r, carry):
        for j in range(_D // _LANES):
            zbuf[r, pl.ds(j * _LANES, _LANES)] = zv
        return carry

    lax.fori_loop(0, _XB, _zrow, 0)
    start = s * _RPT

    def _zc(q, carry):
        pltpu.sync_copy(zbuf, agg_sh.at[pl.ds(start + q * _XB, _XB)])
        return carry

    lax.fori_loop(0, _RPT // _XB, _zc, 0)

    @pl.when(s == _NS - 1)
    def _():
        pltpu.sync_copy(zbuf, agg_sh.at[pl.ds(_NS * _RPT, _XB)])

    plsc.subcore_barrier()

    base = tid * (_CH * _B)

    def _iblock(g, carry):
        pltpu.sync_copy(src_hbm.at[tid, g], src_v)
        pltpu.sync_copy(dst_hbm.at[tid, g], dst_v)

        def _chunk(k, kcarry):
            cp_g = pltpu.async_copy(h_hbm.at[src_v.at[k]], rows_v, sem_g)
            cp_e = pltpu.async_copy(
                e_hbm.at[pl.ds(base + (g * _IG + k) * _B, _B)], msg_v, sem_e)
            cp_g.wait()
            cp_e.wait()

            def _row(r, rcarry):
                for j in range(_D // _LANES):
                    sl = pl.ds(j * _LANES, _LANES)
                    msg_v[r, sl] = jnp.maximum(msg_v[r, sl] + rows_v[r, sl], 0.0)
                return rcarry

            lax.fori_loop(0, _B, _row, 0)
            pltpu.sync_copy(msg_v, agg_sh.at[dst_v.at[k]], add=True)
            return kcarry

        lax.fori_loop(0, _IG, _chunk, 0)
        return carry

    lax.fori_loop(0, _CH // _IG, _iblock, 0)
    plsc.subcore_barrier()

    def _xc(q, carry):
        r0 = start + q * _XB
        pltpu.sync_copy(agg_sh.at[pl.ds(r0, _XB)], zbuf)
        pltpu.sync_copy(zbuf, out_hbm.at[c, pl.ds(r0, _XB)])
        return carry

    lax.fori_loop(0, _RPT // _XB, _xc, 0)

    @pl.when(s == _NS - 1)
    def _():
        r0 = _NS * _RPT
        pltpu.sync_copy(agg_sh.at[pl.ds(r0, _XB)], zbuf)
        pltpu.sync_copy(zbuf, out_hbm.at[c, pl.ds(r0, _XB)])


_sc_agg = pl.kernel(
    _sc_agg_body,
    out_type=jax.ShapeDtypeStruct((_NC, _N, _D), jnp.float32),
    mesh=plsc.VectorSubcoreMesh(core_axis_name="c", subcore_axis_name="s"),
    scratch_types=[
        pltpu.VMEM((_IG, _B), jnp.int32),
        pltpu.VMEM((_IG, _B), jnp.int32),
        pltpu.VMEM((_B, _D), jnp.float32),
        pltpu.VMEM((_B, _D), jnp.float32),
        pltpu.VMEM((_XB, _D), jnp.float32),
        pltpu.VMEM_SHARED((_N, _D), jnp.float32),
        pltpu.SemaphoreType.DMA,
        pltpu.SemaphoreType.DMA,
    ],
)


_NB = 1000
_EB = 4000


def _enc_body(x_ref, w_ref, b_ref, o_ref):
    o_ref[...] = jnp.dot(x_ref[...], w_ref[...],
                         preferred_element_type=jnp.float32) + b_ref[...]


_enc_call = pl.pallas_call(
    _enc_body,
    grid=(_N // _NB,),
    in_specs=[
        pl.BlockSpec((_NB, _D), lambda i: (i, 0)),
        pl.BlockSpec((_D, _D), lambda i: (0, 0)),
        pl.BlockSpec((1, _D), lambda i: (0, 0)),
    ],
    out_specs=pl.BlockSpec((_NB, _D), lambda i: (i, 0)),
    out_shape=jax.ShapeDtypeStruct((_N, _D), jnp.float32),
)


def _edge_body(a_ref, w_ref, b_ref, o_ref):
    o_ref[...] = jnp.dot(a_ref[...], w_ref[...],
                         preferred_element_type=jnp.float32) + b_ref[...]


_edge_call = pl.pallas_call(
    _edge_body,
    grid=(_E // _EB,),
    in_specs=[
        pl.BlockSpec((_EB, _EDGE_D), lambda i: (i, 0)),
        pl.BlockSpec((_EDGE_D, _D), lambda i: (0, 0)),
        pl.BlockSpec((1, _D), lambda i: (0, 0)),
    ],
    out_specs=pl.BlockSpec((_EB, _D), lambda i: (i, 0)),
    out_shape=jax.ShapeDtypeStruct((_E, _D), jnp.float32),
)


def _mlp_body(h_ref, a_ref, s_ref, w1_ref, b1_ref, w2_ref, b2_ref, o_ref):
    scale = s_ref[0, 0]
    z = h_ref[...] * scale + a_ref[0] + a_ref[1]
    z = jnp.maximum(
        jnp.dot(z, w1_ref[...], preferred_element_type=jnp.float32)
        + b1_ref[...], 0.0)
    o_ref[...] = jnp.maximum(
        jnp.dot(z, w2_ref[...], preferred_element_type=jnp.float32)
        + b2_ref[...], 0.0)


_mlp_call = pl.pallas_call(
    _mlp_body,
    grid=(_N // _NB,),
    in_specs=[
        pl.BlockSpec((_NB, _D), lambda i: (i, 0)),
        pl.BlockSpec((_NC, _NB, _D), lambda i: (0, i, 0)),
        pl.BlockSpec((1, 1), lambda i: (0, 0)),
        pl.BlockSpec((_D, _D), lambda i: (0, 0)),
        pl.BlockSpec((1, _D), lambda i: (0, 0)),
        pl.BlockSpec((_D, _D), lambda i: (0, 0)),
        pl.BlockSpec((1, _D), lambda i: (0, 0)),
    ],
    out_specs=pl.BlockSpec((_NB, _D), lambda i: (i, 0)),
    out_shape=jax.ShapeDtypeStruct((_N, _D), jnp.float32),
)


def _pool_body(h_ref, bt_ref, wfc_ref, bfc_ref, o_ref):
    i = pl.program_id(0)
    bv = bt_ref[...].reshape(1, _NB)
    gid = lax.broadcasted_iota(jnp.int32, (_G, _NB), 0)
    oh = (gid == bv).astype(jnp.float32)
    gp = jnp.dot(oh, h_ref[...], preferred_element_type=jnp.float32)
    contrib = jnp.dot(gp, wfc_ref[...], preferred_element_type=jnp.float32)

    @pl.when(i == 0)
    def _():
        o_ref[...] = contrib + bfc_ref[...]

    @pl.when(i != 0)
    def _():
        o_ref[...] += contrib


_pool_call = pl.pallas_call(
    _pool_body,
    grid=(_N // _NB,),
    in_specs=[
        pl.BlockSpec((_NB, _D), lambda i: (i, 0)),
        pl.BlockSpec((1, 1, _NB), lambda i: (i, 0, 0)),
        pl.BlockSpec((_D, _OUT), lambda i: (0, 0)),
        pl.BlockSpec((1, _OUT), lambda i: (0, 0)),
    ],
    out_specs=pl.BlockSpec((_G, _OUT), lambda i: (0, 0)),
    out_shape=jax.ShapeDtypeStruct((_G, _OUT), jnp.float32),
)


def kernel(x, edge_index, edge_attr, batch, W_enc, b_enc, eps_all,
           W_edge_all, b_edge_all, W1_all, b1_all, W2_all, b2_all,
           W_fc, b_fc):
    src = edge_index[0].astype(jnp.int32)
    dst = edge_index[1].astype(jnp.int32)
    src_r = src.reshape(_NT, _CH // _IG, _IG, _B)
    dst_r = dst.reshape(_NT, _CH // _IG, _IG, _B)
    batch_r = batch.astype(jnp.int32).reshape(_N // _NB, 1, _NB)

    h = _enc_call(x, W_enc, b_enc.reshape(1, _D))
    for i in range(_L):
        e = _edge_call(edge_attr, W_edge_all[i], b_edge_all[i].reshape(1, _D))
        agg = _sc_agg(h, e, src_r, dst_r)
        h = _mlp_call(h, agg, (1.0 + eps_all[i]).reshape(1, 1),
                      W1_all[i], b1_all[i].reshape(1, _D),
                      W2_all[i], b2_all[i].reshape(1, _D))
    return _pool_call(h, batch_r, W_fc, b_fc.reshape(1, _OUT))


# speedup vs baseline: 4.8672x; 1.3605x over previous
import functools

import jax
import jax.numpy as jnp
from jax import lax
from jax.experimental import pallas as pl
from jax.experimental.pallas import tpu as pltpu
from jax.experimental.pallas import tpu_sc as plsc

_N = 10000
_E = 320000
_D = 128
_EDGE_D = 16
_G = 64
_L = 3
_OUT = 128

_NC = 2
_NS = 16
_NT = _NC * _NS
_B = 80
_CH = _E // (_NT * _B)
_IG = 25
_RPT = 624
_XB = 16
_LANES = 16


def _sc_agg_body(h_hbm, e_hbm, src_hbm, dst_hbm, out_hbm,
                 src_v, dst_v, rows_a, rows_b, msg_a, msg_b, agg_sh,
                 sga, sgb, sea, seb):
    c = lax.axis_index("c")
    s = lax.axis_index("s")
    tid = c * _NS + s

    zv = jnp.zeros((_LANES,), jnp.float32)

    def _zrow(r, carry):
        for j in range(_D // _LANES):
            rows_a[r, pl.ds(j * _LANES, _LANES)] = zv
        return carry

    lax.fori_loop(0, _B, _zrow, 0)
    start = s * _RPT
    for q in range(7):
        pltpu.sync_copy(rows_a, agg_sh.at[pl.ds(start + q * _B, _B)])

    @pl.when(s < _NS - 1)
    def _():
        pltpu.sync_copy(rows_a.at[pl.ds(0, 64)],
                        agg_sh.at[pl.ds(start + 7 * _B, 64)])

    @pl.when(s == _NS - 1)
    def _():
        pltpu.sync_copy(rows_a, agg_sh.at[pl.ds(start + 7 * _B, _B)])

    plsc.subcore_barrier()

    base = tid * (_CH * _B)

    def _issue(g, k, rows_ref, msg_ref, sg, se):
        pltpu.async_copy(h_hbm.at[src_v.at[k]], rows_ref, sg)
        pltpu.async_copy(
            e_hbm.at[pl.ds(base + (g * _IG + k) * _B, _B)], msg_ref, se)

    def _drain(g, k, rows_ref, msg_ref, sg, se):
        pltpu.make_async_copy(h_hbm.at[src_v.at[k]], rows_ref, sg).wait()
        pltpu.make_async_copy(
            e_hbm.at[pl.ds(base + (g * _IG + k) * _B, _B)], msg_ref, se).wait()

        def _row(r, rcarry):
            for j in range(_D // _LANES):
                sl = pl.ds(j * _LANES, _LANES)
                msg_ref[r, sl] = jnp.maximum(msg_ref[r, sl] + rows_ref[r, sl],
                                             0.0)
            return rcarry

        lax.fori_loop(0, _B, _row, 0)
        pltpu.sync_copy(msg_ref, agg_sh.at[dst_v.at[k]], add=True)

    def _iblock(g, carry):
        pltpu.sync_copy(src_hbm.at[tid, g], src_v)
        pltpu.sync_copy(dst_hbm.at[tid, g], dst_v)
        _issue(g, 0, rows_a, msg_a, sga, sea)

        def _pair(m, kcarry):
            k0 = 2 * m
            _issue(g, k0 + 1, rows_b, msg_b, sgb, seb)
            _drain(g, k0, rows_a, msg_a, sga, sea)
            _issue(g, k0 + 2, rows_a, msg_a, sga, sea)
            _drain(g, k0 + 1, rows_b, msg_b, sgb, seb)
            return kcarry

        lax.fori_loop(0, (_IG - 1) // 2, _pair, 0)
        _drain(g, _IG - 1, rows_a, msg_a, sga, sea)
        return carry

    lax.fori_loop(0, _CH // _IG, _iblock, 0)
    plsc.subcore_barrier()

    for q in range(7):
        r0 = start + q * _B
        pltpu.sync_copy(agg_sh.at[pl.ds(r0, _B)], rows_a)
        pltpu.sync_copy(rows_a, out_hbm.at[c, pl.ds(r0, _B)])

    @pl.when(s < _NS - 1)
    def _():
        r0 = start + 7 * _B
        pltpu.sync_copy(agg_sh.at[pl.ds(r0, 64)], rows_a.at[pl.ds(0, 64)])
        pltpu.sync_copy(rows_a.at[pl.ds(0, 64)], out_hbm.at[c, pl.ds(r0, 64)])

    @pl.when(s == _NS - 1)
    def _():
        r0 = start + 7 * _B
        pltpu.sync_copy(agg_sh.at[pl.ds(r0, _B)], rows_a)
        pltpu.sync_copy(rows_a, out_hbm.at[c, pl.ds(r0, _B)])


_sc_agg = pl.kernel(
    _sc_agg_body,
    out_type=jax.ShapeDtypeStruct((_NC, _N, _D), jnp.float32),
    mesh=plsc.VectorSubcoreMesh(core_axis_name="c", subcore_axis_name="s"),
    scratch_types=[
        pltpu.VMEM((_IG, _B), jnp.int32),
        pltpu.VMEM((_IG, _B), jnp.int32),
        pltpu.VMEM((_B, _D), jnp.float32),
        pltpu.VMEM((_B, _D), jnp.float32),
        pltpu.VMEM((_B, _D), jnp.float32),
        pltpu.VMEM((_B, _D), jnp.float32),
        pltpu.VMEM_SHARED((_N, _D), jnp.float32),
        pltpu.SemaphoreType.DMA,
        pltpu.SemaphoreType.DMA,
        pltpu.SemaphoreType.DMA,
        pltpu.SemaphoreType.DMA,
    ],
)


_NB = 1000
_EB = 4000


def _enc_body(x_ref, w_ref, b_ref, o_ref):
    o_ref[...] = jnp.dot(x_ref[...], w_ref[...],
                         preferred_element_type=jnp.float32) + b_ref[...]


_enc_call = pl.pallas_call(
    _enc_body,
    grid=(_N // _NB,),
    in_specs=[
        pl.BlockSpec((_NB, _D), lambda i: (i, 0)),
        pl.BlockSpec((_D, _D), lambda i: (0, 0)),
        pl.BlockSpec((1, _D), lambda i: (0, 0)),
    ],
    out_specs=pl.BlockSpec((_NB, _D), lambda i: (i, 0)),
    out_shape=jax.ShapeDtypeStruct((_N, _D), jnp.float32),
)


def _edge_body(a_ref, w_ref, b_ref, o_ref):
    o_ref[...] = jnp.dot(a_ref[...], w_ref[...],
                         preferred_element_type=jnp.float32) + b_ref[...]


_edge_call = pl.pallas_call(
    _edge_body,
    grid=(_E // _EB,),
    in_specs=[
        pl.BlockSpec((_EB, _EDGE_D), lambda i: (i, 0)),
        pl.BlockSpec((_EDGE_D, _D), lambda i: (0, 0)),
        pl.BlockSpec((1, _D), lambda i: (0, 0)),
    ],
    out_specs=pl.BlockSpec((_EB, _D), lambda i: (i, 0)),
    out_shape=jax.ShapeDtypeStruct((_E, _D), jnp.float32),
)


def _mlp_body(h_ref, a_ref, s_ref, w1_ref, b1_ref, w2_ref, b2_ref, o_ref):
    scale = s_ref[0, 0]
    z = h_ref[...] * scale + a_ref[0] + a_ref[1]
    z = jnp.maximum(
        jnp.dot(z, w1_ref[...], preferred_element_type=jnp.float32)
        + b1_ref[...], 0.0)
    o_ref[...] = jnp.maximum(
        jnp.dot(z, w2_ref[...], preferred_element_type=jnp.float32)
        + b2_ref[...], 0.0)


_mlp_call = pl.pallas_call(
    _mlp_body,
    grid=(_N // _NB,),
    in_specs=[
        pl.BlockSpec((_NB, _D), lambda i: (i, 0)),
        pl.BlockSpec((_NC, _NB, _D), lambda i: (0, i, 0)),
        pl.BlockSpec((1, 1), lambda i: (0, 0)),
        pl.BlockSpec((_D, _D), lambda i: (0, 0)),
        pl.BlockSpec((1, _D), lambda i: (0, 0)),
        pl.BlockSpec((_D, _D), lambda i: (0, 0)),
        pl.BlockSpec((1, _D), lambda i: (0, 0)),
    ],
    out_specs=pl.BlockSpec((_NB, _D), lambda i: (i, 0)),
    out_shape=jax.ShapeDtypeStruct((_N, _D), jnp.float32),
)


def _pool_body(h_ref, bt_ref, wfc_ref, bfc_ref, o_ref):
    i = pl.program_id(0)
    bv = bt_ref[...].reshape(1, _NB)
    gid = lax.broadcasted_iota(jnp.int32, (_G, _NB), 0)
    oh = (gid == bv).astype(jnp.float32)
    gp = jnp.dot(oh, h_ref[...], preferred_element_type=jnp.float32)
    contrib = jnp.dot(gp, wfc_ref[...], preferred_element_type=jnp.float32)

    @pl.when(i == 0)
    def _():
        o_ref[...] = contrib + bfc_ref[...]

    @pl.when(i != 0)
    def _():
        o_ref[...] += contrib


_pool_call = pl.pallas_call(
    _pool_body,
    grid=(_N // _NB,),
    in_specs=[
        pl.BlockSpec((_NB, _D), lambda i: (i, 0)),
        pl.BlockSpec((1, 1, _NB), lambda i: (i, 0, 0)),
        pl.BlockSpec((_D, _OUT), lambda i: (0, 0)),
        pl.BlockSpec((1, _OUT), lambda i: (0, 0)),
    ],
    out_specs=pl.BlockSpec((_G, _OUT), lambda i: (0, 0)),
    out_shape=jax.ShapeDtypeStruct((_G, _OUT), jnp.float32),
)


def kernel(x, edge_index, edge_attr, batch, W_enc, b_enc, eps_all,
           W_edge_all, b_edge_all, W1_all, b1_all, W2_all, b2_all,
           W_fc, b_fc):
    src = edge_index[0].astype(jnp.int32)
    dst = edge_index[1].astype(jnp.int32)
    src_r = src.reshape(_NT, _CH // _IG, _IG, _B)
    dst_r = dst.reshape(_NT, _CH // _IG, _IG, _B)
    batch_r = batch.astype(jnp.int32).reshape(_N // _NB, 1, _NB)

    h = _enc_call(x, W_enc, b_enc.reshape(1, _D))
    for i in range(_L):
        e = _edge_call(edge_attr, W_edge_all[i], b_edge_all[i].reshape(1, _D))
        agg = _sc_agg(h, e, src_r, dst_r)
        h = _mlp_call(h, agg, (1.0 + eps_all[i]).reshape(1, 1),
                      W1_all[i], b1_all[i].reshape(1, _D),
                      W2_all[i], b2_all[i].reshape(1, _D))
    return _pool_call(h, batch_r, W_fc, b_fc.reshape(1, _OUT))
